# baseline (speedup 1.0000x reference)
import functools

import jax
import jax.numpy as jnp
from jax import lax
from jax.experimental import pallas as pl
from jax.experimental.pallas import tpu as pltpu
from jax.experimental.pallas import tpu_sc as plsc

_NUM_EMBEDDINGS = 1024
_EMBEDDING_DIM = 64
_BETA = 0.25
_N_ROWS = 16 * 1024
_BLOCK = 2048
_GRID = _N_ROWS // _BLOCK

_SC_CORES = 2
_SC_SUBCORES = 16
_SC_WORKERS = _SC_CORES * _SC_SUBCORES
_ROWS_PER_WORKER = _N_ROWS // _SC_WORKERS
_PAD_DIM = 128


def _vq_block(x_ref, c_ref, idx_ref, counts_ref, loss_ref, perp_ref):
    g = pl.program_id(0)

    x = x_ref[...]
    c = c_ref[...]

    xx = jnp.sum(x * x, axis=1, keepdims=True)
    cc = jnp.sum(c * c, axis=0, keepdims=True)
    m = jnp.dot(x.astype(jnp.bfloat16), c.astype(jnp.bfloat16),
                preferred_element_type=jnp.float32)
    d = (xx - 2.0 * m) + cc

    dmin = jnp.min(d, axis=1, keepdims=True)
    onehot = (d == dmin).astype(jnp.float32)

    iota_row = lax.broadcasted_iota(
        jnp.int32, d.shape, 1).astype(jnp.float32)
    idx_f = jnp.sum(onehot * iota_row, axis=1, keepdims=True)
    idx = jnp.minimum(idx_f, float(_NUM_EMBEDDINGS - 1)).astype(jnp.int32)
    idx_ref[...] = idx

    @pl.when(g == 0)
    def _init():
        counts_ref[...] = jnp.zeros_like(counts_ref)
        loss_ref[0, 0] = 0.0
        perp_ref[0, 0] = 0.0

    counts_ref[...] += jnp.sum(onehot, axis=0, keepdims=True)
    loss_ref[0, 0] += jnp.sum(dmin)

    @pl.when(g == _GRID - 1)
    def _finalize():
        loss_ref[0, 0] = loss_ref[0, 0] * (_BETA / (_N_ROWS * _EMBEDDING_DIM))
        p = counts_ref[...] * (1.0 / _N_ROWS)
        ent = -jnp.sum(p * jnp.log(p + 1e-10))
        perp_ref[0, 0] = jnp.exp(ent)


@functools.partial(
    pl.kernel,
    mesh=plsc.VectorSubcoreMesh(core_axis_name="c", subcore_axis_name="s",
                                num_cores=_SC_CORES),
    out_type=jax.ShapeDtypeStruct((_N_ROWS, _PAD_DIM), jnp.float32),
    scratch_types=[
        pltpu.VMEM((_ROWS_PER_WORKER,), jnp.int32),
        pltpu.VMEM((_ROWS_PER_WORKER, _PAD_DIM), jnp.float32),
        pltpu.SemaphoreType.DMA,
    ],
)
def _sc_gather(ct_hbm, idx_hbm, out_hbm, idx_v, rows_v, sem):
    wid = lax.axis_index("s") * _SC_CORES + lax.axis_index("c")
    base = wid * _ROWS_PER_WORKER
    pltpu.sync_copy(idx_hbm.at[pl.ds(base, _ROWS_PER_WORKER)], idx_v)
    copies = []
    for k in range(_ROWS_PER_WORKER // 128):
        copies.append(pltpu.async_copy(
            ct_hbm.at[idx_v.at[pl.ds(k * 128, 128)]],
            rows_v.at[pl.ds(k * 128, 128)], sem))
    for cp in copies:
        cp.wait()
    pltpu.sync_copy(rows_v, out_hbm.at[pl.ds(base, _ROWS_PER_WORKER)])


@jax.jit
def _vq_forward(flat_inputs, codebook, codebook_t):
    idx, counts, loss, perp = pl.pallas_call(
        _vq_block,
        grid=(_GRID,),
        in_specs=[
            pl.BlockSpec((_BLOCK, _EMBEDDING_DIM), lambda g: (g, 0)),
            pl.BlockSpec((_EMBEDDING_DIM, _NUM_EMBEDDINGS), lambda g: (0, 0)),
        ],
        out_specs=[
            pl.BlockSpec((_BLOCK, 1), lambda g: (g, 0)),
            pl.BlockSpec((1, _NUM_EMBEDDINGS), lambda g: (0, 0)),
            pl.BlockSpec(memory_space=pltpu.SMEM),
            pl.BlockSpec(memory_space=pltpu.SMEM),
        ],
        out_shape=[
            jax.ShapeDtypeStruct((_N_ROWS, 1), jnp.int32),
            jax.ShapeDtypeStruct((1, _NUM_EMBEDDINGS), jnp.float32),
            jax.ShapeDtypeStruct((1, 1), jnp.float32),
            jax.ShapeDtypeStruct((1, 1), jnp.float32),
        ],
    )(flat_inputs, codebook)
    q_pad = _sc_gather(codebook_t, jnp.reshape(idx, (_N_ROWS,)))
    return q_pad[:, :_EMBEDDING_DIM], loss[0, 0], perp[0, 0]


def kernel(inputs, codebook, training=True):
    flat_inputs = jnp.reshape(inputs, (-1, _EMBEDDING_DIM))
    ct_pad = jnp.pad(codebook.T, ((0, 0), (0, _PAD_DIM - _EMBEDDING_DIM)))
    q, loss, perp = _vq_forward(flat_inputs, codebook, ct_pad)
    ste = jnp.reshape(q, inputs.shape)
    return ste, perp, loss
